# baseline (speedup 1.0000x reference)
import jax
import jax.numpy as jnp
from jax.experimental import pallas as pl


def _final_linear_kernel(h_ref, w_ref, b_ref, o_ref):
    o_ref[...] = h_ref[...] @ w_ref[...] + b_ref[...]


def _gatv2_layer(x, src, dst, edge_attr, p, heads, out_ch, n_nodes):
    xl = (x @ p["Wl"] + p["bl"]).reshape(n_nodes, heads, out_ch)
    xr = (x @ p["Wr"] + p["br"]).reshape(n_nodes, heads, out_ch)
    e = (edge_attr @ p["We"]).reshape(-1, heads, out_ch)
    xj = xl[src]
    m = xj + xr[dst] + e
    m = jax.nn.leaky_relu(m, negative_slope=0.2)
    alpha = (m * p["att"][None]).sum(-1)
    amax = jax.ops.segment_max(alpha, dst, num_segments=n_nodes)
    amax = jnp.where(jnp.isfinite(amax), amax, 0.0)
    ex = jnp.exp(alpha - amax[dst])
    denom = jax.ops.segment_sum(ex, dst, num_segments=n_nodes)
    a = ex / (denom[dst] + 1e-16)
    out = jax.ops.segment_sum(xj * a[..., None], dst, num_segments=n_nodes)
    return out.reshape(n_nodes, heads * out_ch) + p["bias"]


def kernel(x, edge_index, edge_attr, params):
    src, dst = edge_index[0], edge_index[1]
    n = x.shape[0]
    h = _gatv2_layer(x, src, dst, edge_attr, params["c1"], 8, 16, n)
    h = jax.nn.elu(h)
    h = _gatv2_layer(h, src, dst, edge_attr, params["c2"], 8, 16, n)
    h = jax.nn.elu(h)
    h = _gatv2_layer(h, src, dst, edge_attr, params["c3"], 8, 16, n)
    h = jax.nn.elu(h)
    h = _gatv2_layer(h, src, dst, edge_attr, params["c4"], 8, 8, n)
    out = pl.pallas_call(
        _final_linear_kernel,
        out_shape=jax.ShapeDtypeStruct((n, 1), jnp.float32),
    )(h, params["lin_W"], params["lin_b"])
    return out


# speedup vs baseline: 15.3195x; 15.3195x over previous
import functools

import jax
import jax.numpy as jnp
from jax import lax
from jax.experimental import pallas as pl
from jax.experimental.pallas import tpu as pltpu
from jax.experimental.pallas import tpu_sc as plsc

N = 50000
E = 800000
NW = 32
NB = 128
ECH = 256
RING = 2 * ECH
MM_BLK = 1000

def _lane():
    return lax.iota(jnp.int32, 16)


def _matmul2_kernel(x_ref, wl_ref, bl_ref, wr_ref, br_ref, xl_ref, xr_ref,
                    *, act):
    x = x_ref[...]
    if act == "elu":
        x = jnp.where(x > 0, x, jnp.exp(x) - 1.0)
    xl_ref[...] = jnp.dot(x, wl_ref[...],
                          preferred_element_type=jnp.float32) + bl_ref[...]
    xr_ref[...] = jnp.dot(x, wr_ref[...],
                          preferred_element_type=jnp.float32) + br_ref[...]


def _dense_xlxr(x, wl, bl, wr, br, act):
    n, k = x.shape
    f = wl.shape[1]
    return pl.pallas_call(
        functools.partial(_matmul2_kernel, act=act),
        grid=(n // MM_BLK,),
        in_specs=[
            pl.BlockSpec((MM_BLK, k), lambda i: (i, 0)),
            pl.BlockSpec((k, f), lambda i: (0, 0)),
            pl.BlockSpec((f,), lambda i: (0,)),
            pl.BlockSpec((k, f), lambda i: (0, 0)),
            pl.BlockSpec((f,), lambda i: (0,)),
        ],
        out_specs=[
            pl.BlockSpec((MM_BLK, f), lambda i: (i, 0)),
            pl.BlockSpec((MM_BLK, f), lambda i: (i, 0)),
        ],
        out_shape=[
            jax.ShapeDtypeStruct((n + NB, f), jnp.float32),
            jax.ShapeDtypeStruct((n + NB, f), jnp.float32),
        ],
    )(x, wl, bl, wr, br)


def _final_kernel(h_ref, w_ref, b_ref, o_ref):
    o_ref[...] = jnp.dot(h_ref[...], w_ref[...],
                         preferred_element_type=jnp.float32) + b_ref[...]


def _final_linear(h, w, b):
    n, k = h.shape
    return pl.pallas_call(
        _final_kernel,
        grid=(n // MM_BLK,),
        in_specs=[
            pl.BlockSpec((MM_BLK, k), lambda i: (i, 0)),
            pl.BlockSpec((k, 1), lambda i: (0, 0)),
            pl.BlockSpec((1,), lambda i: (0,)),
        ],
        out_specs=pl.BlockSpec((MM_BLK, 1), lambda i: (i, 0)),
        out_shape=jax.ShapeDtypeStruct((n, 1), jnp.float32),
    )(h, w, b)


def _extract_i(ref, j):
    base = pl.multiple_of((j // 16) * 16, 16)
    v = ref[pl.ds(base, 16)]
    return jnp.sum(jnp.where(_lane() == (j - base), v, 0))


def _extract_f(ref, j):
    base = pl.multiple_of((j // 16) * 16, 16)
    v = ref[pl.ds(base, 16)]
    return jnp.sum(jnp.where(_lane() == (j - base), v, 0.0))


def _make_edge_kernel(nvec, hpv):
    def body(xl_hbm, xr_hbm, src_hbm, ea_hbm, rs_hbm, nstart_hbm,
             we_hbm, att_hbm, bias_hbm, out_hbm,
             ns_v, rs0_v, rs_blk, xr_blk, out_blk, src_ring, ea_ring,
             xj_ring, we_v, att_v, bias_v, gsem):
        wid = lax.axis_index("s") * 2 + lax.axis_index("c")

        pltpu.sync_copy(we_hbm, we_v)
        pltpu.sync_copy(att_hbm, att_v)
        pltpu.sync_copy(bias_hbm, bias_v)
        pltpu.sync_copy(nstart_hbm, ns_v)
        ns = _extract_i(ns_v, wid)
        ne = _extract_i(ns_v, wid + 1)
        nnodes = ne - ns

        @pl.when(nnodes > 0)
        def _worker():
            rsb = pl.multiple_of((ns // 8) * 8, 8)
            pltpu.sync_copy(rs_hbm.at[pl.ds(rsb, 16)], rs0_v)
            eb = _extract_i(rs0_v, ns - rsb)
            cb0 = pl.multiple_of((eb // 8) * 8, 8)

            def stage(i):
                slot = pl.multiple_of((i % 2) * ECH, 8)
                base = pl.multiple_of(cb0 + i * ECH, 8)
                pltpu.sync_copy(src_hbm.at[pl.ds(base, ECH)],
                                src_ring.at[pl.ds(slot, ECH)])
                pltpu.sync_copy(ea_hbm.at[pl.ds(base, ECH)],
                                ea_ring.at[pl.ds(slot, ECH)])
                return pltpu.async_copy(
                    xl_hbm.at[src_ring.at[pl.ds(slot, ECH)]],
                    xj_ring.at[pl.ds(slot, ECH)], gsem)

            def drain_one():
                pltpu.make_async_copy(
                    xl_hbm.at[pl.ds(0, ECH)],
                    xj_ring.at[pl.ds(0, ECH)], gsem).wait()

            stage(0)
            drain_one()
            stage(1)

            zero16 = jnp.zeros((16,), jnp.float32)
            nblk = (nnodes + (NB - 1)) // NB

            def blk_body(b, ci):
                nbase = ns + b * NB
                nb_al = pl.multiple_of((nbase // 8) * 8, 8)
                d0 = nbase - nb_al
                pltpu.sync_copy(xr_hbm.at[pl.ds(nb_al, NB + 8)], xr_blk)
                pltpu.sync_copy(rs_hbm.at[pl.ds(nb_al, NB + 16)], rs_blk)

                def node_body(nloc, ci):
                    s = _extract_i(rs_blk, d0 + nloc)
                    t = _extract_i(rs_blk, d0 + nloc + 1)

                    def edge_body(e, carry):
                        ci = carry[0]
                        accs = carry[1:]
                        need = (e - cb0) // ECH

                        @pl.when(need != ci)
                        def _advance():
                            drain_one()
                            stage(ci + 2)

                        iring = (e - cb0) % RING
                        ea_s = _extract_f(ea_ring, iring)
                        new_num = []
                        exs = []
                        for v in range(nvec):
                            sl = pl.ds(v * 16, 16)
                            xj = xj_ring[iring, sl]
                            m = (xj + xr_blk[d0 + nloc, sl]
                                 + ea_s * we_v[pl.ds(v * 16, 16)])
                            m = jnp.maximum(m, 0.2 * m)
                            pre = plsc.cumsum(m * att_v[pl.ds(v * 16, 16)])
                            hi = jnp.sum(jnp.where(_lane() == 15, pre, 0.0))
                            if hpv == 1:
                                a = jnp.full((16,), hi)
                            else:
                                lo = jnp.sum(
                                    jnp.where(_lane() == 7, pre, 0.0))
                                a = jnp.where(_lane() < 8, lo, hi - lo)
                            exb = jnp.exp(a)
                            exs.append(exb)
                            new_num.append(accs[v] + exb * xj)
                        new_den = [accs[nvec + v] + exs[v]
                                   for v in range(nvec)]
                        return tuple([need] + new_num + new_den)

                    init = tuple([ci] + [zero16] * (2 * nvec))
                    res = lax.fori_loop(s, t, edge_body, init)
                    for v in range(nvec):
                        sl = pl.ds(v * 16, 16)
                        out_blk[nloc, sl] = (
                            res[1 + v] / (res[1 + nvec + v] + 1e-16)
                            + bias_v[sl])
                    return res[0]

                nmax = jnp.minimum(NB, ne - nbase)
                ci = lax.fori_loop(0, nmax, node_body, ci)

                off = jnp.int32(0)
                for bit in range(7, -1, -1):
                    w = 1 << bit
                    hit = (nmax & w) != 0
                    off_c = off

                    @pl.when(hit)
                    def _flush(off_c=off_c, w=w):
                        pltpu.sync_copy(
                            out_blk.at[pl.ds(off_c, w)],
                            out_hbm.at[pl.ds(nbase + off_c, w)])

                    off = off + jnp.where(hit, w, 0)
                return ci

            lax.fori_loop(0, nblk, blk_body, jnp.int32(0))
            drain_one()

    return body


def _gat_layer_sc(xl, xr, srcs, eas, rs, nstart, wev, attv, biasv,
                  heads, out_ch):
    f = heads * out_ch
    nvec = f // 16
    hpv = 16 // out_ch
    mesh = plsc.VectorSubcoreMesh(core_axis_name="c", subcore_axis_name="s")
    kern = pl.kernel(
        _make_edge_kernel(nvec, hpv),
        out_type=jax.ShapeDtypeStruct((N + NB, nvec * 16), jnp.float32),
        mesh=mesh,
        compiler_params=pltpu.CompilerParams(
            needs_layout_passes=False, use_tc_tiling_on_sc=False),
        scratch_types=[
            pltpu.VMEM((48,), jnp.int32),
            pltpu.VMEM((16,), jnp.int32),
            pltpu.VMEM((NB + 16,), jnp.int32),
            pltpu.VMEM((NB + 8, nvec * 16), jnp.float32),
            pltpu.VMEM((NB, nvec * 16), jnp.float32),
            pltpu.VMEM((RING,), jnp.int32),
            pltpu.VMEM((RING,), jnp.float32),
            pltpu.VMEM((RING, nvec * 16), jnp.float32),
            pltpu.VMEM((nvec * 16,), jnp.float32),
            pltpu.VMEM((nvec * 16,), jnp.float32),
            pltpu.VMEM((nvec * 16,), jnp.float32),
            pltpu.SemaphoreType.DMA,
        ],
    )
    out = kern(xl, xr, srcs, eas, rs, nstart, wev, attv, biasv)
    return out[:N]


def kernel(x, edge_index, edge_attr, params):
    src = edge_index[0]
    dst = edge_index[1]
    ea = edge_attr[:, 0]

    perm = jnp.argsort(dst)
    sdst = dst[perm]
    srcs = src[perm]
    eas = ea[perm]
    row_start = jnp.searchsorted(
        sdst, jnp.arange(N + 1, dtype=jnp.int32)).astype(jnp.int32)
    targets = (jnp.arange(NW + 1, dtype=jnp.int32) *
               jnp.int32(E // NW)).astype(jnp.int32)
    nstart = jnp.searchsorted(row_start, targets).astype(jnp.int32)
    nstart = nstart.at[NW].set(N)

    srcs_p = jnp.pad(srcs, (0, 4 * ECH))
    eas_p = jnp.pad(eas, (0, 4 * ECH))
    rs_p = jnp.pad(row_start, (0, NB + 32), constant_values=E)
    nstart_p = jnp.pad(nstart, (0, 48 - (NW + 1)))

    def layer(h, p, heads, out_ch, act):
        xl, xr = _dense_xlxr(h, p["Wl"], p["bl"], p["Wr"], p["br"], act)
        return _gat_layer_sc(
            xl, xr, srcs_p, eas_p, rs_p, nstart_p,
            p["We"][0], p["att"].reshape(-1), p["bias"],
            heads, out_ch)

    x16 = jnp.pad(x, ((0, 0), (0, 1)))
    p1 = dict(params["c1"])
    p1["Wl"] = jnp.pad(params["c1"]["Wl"], ((0, 1), (0, 0)))
    p1["Wr"] = jnp.pad(params["c1"]["Wr"], ((0, 1), (0, 0)))

    h = layer(x16, p1, 8, 16, None)
    h = layer(h, params["c2"], 8, 16, "elu")
    h = layer(h, params["c3"], 8, 16, "elu")
    h = layer(h, params["c4"], 8, 8, "elu")
    return _final_linear(h, params["lin_W"], params["lin_b"])


# speedup vs baseline: 93.1631x; 6.0813x over previous
import functools

import jax
import jax.numpy as jnp
from jax import lax
from jax.experimental import pallas as pl
from jax.experimental.pallas import tpu as pltpu
from jax.experimental.pallas import tpu_sc as plsc

N = 50000
E = 800000
NW = 32
NB = 128
ECH = 256
RING = 2 * ECH
MM_BLK = 1000

def _lane():
    return lax.iota(jnp.int32, 16)


def _matmul2_kernel(x_ref, wl_ref, bl_ref, wr_ref, br_ref, xl_ref, xr_ref,
                    *, act):
    x = x_ref[...]
    if act == "elu":
        x = jnp.where(x > 0, x, jnp.exp(x) - 1.0)
    xl_ref[...] = jnp.dot(x, wl_ref[...],
                          preferred_element_type=jnp.float32) + bl_ref[...]
    xr_ref[...] = jnp.dot(x, wr_ref[...],
                          preferred_element_type=jnp.float32) + br_ref[...]


def _dense_xlxr(x, wl, bl, wr, br, act):
    n, k = x.shape
    f = wl.shape[1]
    return pl.pallas_call(
        functools.partial(_matmul2_kernel, act=act),
        grid=(n // MM_BLK,),
        in_specs=[
            pl.BlockSpec((MM_BLK, k), lambda i: (i, 0)),
            pl.BlockSpec((k, f), lambda i: (0, 0)),
            pl.BlockSpec((f,), lambda i: (0,)),
            pl.BlockSpec((k, f), lambda i: (0, 0)),
            pl.BlockSpec((f,), lambda i: (0,)),
        ],
        out_specs=[
            pl.BlockSpec((MM_BLK, f), lambda i: (i, 0)),
            pl.BlockSpec((MM_BLK, f), lambda i: (i, 0)),
        ],
        out_shape=[
            jax.ShapeDtypeStruct((n + NB, f), jnp.float32),
            jax.ShapeDtypeStruct((n + NB, f), jnp.float32),
        ],
    )(x, wl, bl, wr, br)


def _final_kernel(h_ref, w_ref, b_ref, o_ref):
    o_ref[...] = jnp.dot(h_ref[...], w_ref[...],
                         preferred_element_type=jnp.float32) + b_ref[...]


def _final_linear(h, w, b):
    n, k = h.shape
    return pl.pallas_call(
        _final_kernel,
        grid=(n // MM_BLK,),
        in_specs=[
            pl.BlockSpec((MM_BLK, k), lambda i: (i, 0)),
            pl.BlockSpec((k, 1), lambda i: (0, 0)),
            pl.BlockSpec((1,), lambda i: (0,)),
        ],
        out_specs=pl.BlockSpec((MM_BLK, 1), lambda i: (i, 0)),
        out_shape=jax.ShapeDtypeStruct((n, 1), jnp.float32),
    )(h, w, b)


def _extract_i(ref, j):
    base = pl.multiple_of((j // 16) * 16, 16)
    v = ref[pl.ds(base, 16)]
    return jnp.sum(jnp.where(_lane() == (j - base), v, 0))


def _extract_f(ref, j):
    base = pl.multiple_of((j // 16) * 16, 16)
    v = ref[pl.ds(base, 16)]
    return jnp.sum(jnp.where(_lane() == (j - base), v, 0.0))


def _make_edge_kernel(nvec, hpv):
    def body(xl_hbm, xr_hbm, src_hbm, ea_hbm, rs_hbm, nstart_hbm,
             we_hbm, att_hbm, bias_hbm, out_hbm,
             ns_v, rs0_v, rs_blk, xr_blk, out_blk, src_ring, ea_ring,
             xj_ring, we_v, att_v, bias_v, gsem):
        wid = lax.axis_index("s") * 2 + lax.axis_index("c")

        pltpu.sync_copy(we_hbm, we_v)
        pltpu.sync_copy(att_hbm, att_v)
        pltpu.sync_copy(bias_hbm, bias_v)
        pltpu.sync_copy(nstart_hbm, ns_v)
        ns = _extract_i(ns_v, wid)
        ne = _extract_i(ns_v, wid + 1)
        nnodes = ne - ns

        @pl.when(nnodes > 0)
        def _worker():
            rsb = pl.multiple_of((ns // 8) * 8, 8)
            pltpu.sync_copy(rs_hbm.at[pl.ds(rsb, 16)], rs0_v)
            eb = _extract_i(rs0_v, ns - rsb)
            cb0 = pl.multiple_of((eb // 8) * 8, 8)

            def stage(i):
                slot = pl.multiple_of((i % 2) * ECH, 8)
                base = pl.multiple_of(cb0 + i * ECH, 8)
                pltpu.sync_copy(src_hbm.at[pl.ds(base, ECH)],
                                src_ring.at[pl.ds(slot, ECH)])
                pltpu.sync_copy(ea_hbm.at[pl.ds(base, ECH)],
                                ea_ring.at[pl.ds(slot, ECH)])
                return pltpu.async_copy(
                    xl_hbm.at[src_ring.at[pl.ds(slot, ECH)]],
                    xj_ring.at[pl.ds(slot, ECH)], gsem)

            def drain_one():
                pltpu.make_async_copy(
                    xl_hbm.at[pl.ds(0, ECH)],
                    xj_ring.at[pl.ds(0, ECH)], gsem).wait()

            stage(0)
            drain_one()
            stage(1)

            zero16 = jnp.zeros((16,), jnp.float32)
            nblk = (nnodes + (NB - 1)) // NB

            def blk_body(b, ci):
                nbase = ns + b * NB
                nb_al = pl.multiple_of((nbase // 8) * 8, 8)
                d0 = nbase - nb_al
                pltpu.sync_copy(xr_hbm.at[pl.ds(nb_al, NB + 8)], xr_blk)
                pltpu.sync_copy(rs_hbm.at[pl.ds(nb_al, NB + 16)], rs_blk)

                def node_body(nloc, ci):
                    s = _extract_i(rs_blk, d0 + nloc)
                    t = _extract_i(rs_blk, d0 + nloc + 1)

                    def edge_body(e, carry):
                        ci = carry[0]
                        accs = carry[1:]
                        need = (e - cb0) // ECH

                        @pl.when(need != ci)
                        def _advance():
                            drain_one()
                            stage(ci + 2)

                        iring = (e - cb0) % RING
                        ea_s = _extract_f(ea_ring, iring)
                        new_num = []
                        exs = []
                        for v in range(nvec):
                            sl = pl.ds(v * 16, 16)
                            xj = xj_ring[iring, sl]
                            m = (xj + xr_blk[d0 + nloc, sl]
                                 + ea_s * we_v[pl.ds(v * 16, 16)])
                            m = jnp.maximum(m, 0.2 * m)
                            pre = plsc.cumsum(m * att_v[pl.ds(v * 16, 16)])
                            hi = jnp.sum(jnp.where(_lane() == 15, pre, 0.0))
                            if hpv == 1:
                                a = jnp.full((16,), hi)
                            else:
                                lo = jnp.sum(
                                    jnp.where(_lane() == 7, pre, 0.0))
                                a = jnp.where(_lane() < 8, lo, hi - lo)
                            exb = jnp.exp(a)
                            exs.append(exb)
                            new_num.append(accs[v] + exb * xj)
                        new_den = [accs[nvec + v] + exs[v]
                                   for v in range(nvec)]
                        return tuple([need] + new_num + new_den)

                    init = tuple([ci] + [zero16] * (2 * nvec))
                    res = lax.fori_loop(s, t, edge_body, init)
                    for v in range(nvec):
                        sl = pl.ds(v * 16, 16)
                        out_blk[nloc, sl] = (
                            res[1 + v] / (res[1 + nvec + v] + 1e-16)
                            + bias_v[sl])
                    return res[0]

                nmax = jnp.minimum(NB, ne - nbase)
                ci = lax.fori_loop(0, nmax, node_body, ci)

                off = jnp.int32(0)
                for bit in range(7, -1, -1):
                    w = 1 << bit
                    hit = (nmax & w) != 0
                    off_c = off

                    @pl.when(hit)
                    def _flush(off_c=off_c, w=w):
                        pltpu.sync_copy(
                            out_blk.at[pl.ds(off_c, w)],
                            out_hbm.at[pl.ds(nbase + off_c, w)])

                    off = off + jnp.where(hit, w, 0)
                return ci

            lax.fori_loop(0, nblk, blk_body, jnp.int32(0))
            drain_one()

    return body


def _gat_layer_sc(xl, xr, srcs, eas, rs, nstart, wev, attv, biasv,
                  heads, out_ch):
    f = heads * out_ch
    nvec = f // 16
    hpv = 16 // out_ch
    mesh = plsc.VectorSubcoreMesh(core_axis_name="c", subcore_axis_name="s")
    kern = pl.kernel(
        _make_edge_kernel(nvec, hpv),
        out_type=jax.ShapeDtypeStruct((N + NB, nvec * 16), jnp.float32),
        mesh=mesh,
        compiler_params=pltpu.CompilerParams(
            needs_layout_passes=False, use_tc_tiling_on_sc=False),
        scratch_types=[
            pltpu.VMEM((48,), jnp.int32),
            pltpu.VMEM((16,), jnp.int32),
            pltpu.VMEM((NB + 16,), jnp.int32),
            pltpu.VMEM((NB + 8, nvec * 16), jnp.float32),
            pltpu.VMEM((NB, nvec * 16), jnp.float32),
            pltpu.VMEM((RING,), jnp.int32),
            pltpu.VMEM((RING,), jnp.float32),
            pltpu.VMEM((RING, nvec * 16), jnp.float32),
            pltpu.VMEM((nvec * 16,), jnp.float32),
            pltpu.VMEM((nvec * 16,), jnp.float32),
            pltpu.VMEM((nvec * 16,), jnp.float32),
            pltpu.SemaphoreType.DMA,
        ],
    )
    out = kern(xl, xr, srcs, eas, rs, nstart, wev, attv, biasv)
    return out[:N]


NP = 53248
NCHUNK = NP // 2048
EWP = 25088
EP = NW * EWP
DCH = 1792
DSENT = 52000


def _s1_hist_body(dst_hbm, hist_hbm, hist_v, dst_v):
    wid = lax.axis_index("s") * 2 + lax.axis_index("c")

    def zero_body(i, _):
        hist_v[pl.ds(pl.multiple_of(i * 16, 16), 16)] = (
            jnp.zeros((16,), jnp.int32))
        return 0
    lax.fori_loop(0, NP // 16, zero_body, 0)

    def chunk_body(c, _):
        base = pl.multiple_of(wid * EWP + c * DCH, 8)
        pltpu.sync_copy(dst_hbm.at[pl.ds(base, DCH)], dst_v)

        def vec_body(i, _):
            d16 = dst_v[pl.ds(pl.multiple_of(i * 16, 16), 16)]
            occ, last = plsc.scan_count(d16)
            cur = plsc.load_gather(hist_v, [d16])
            plsc.store_scatter(hist_v, [d16], cur + occ, mask=last)
            return 0
        lax.fori_loop(0, DCH // 16, vec_body, 0)
        return 0
    lax.fori_loop(0, EWP // DCH, chunk_body, 0)
    pltpu.sync_copy(hist_v, hist_hbm.at[wid])


def _s2_prefix_body(hist_hbm, colpre_hbm, csum_hbm, tot_v, acc_v):
    wid = lax.axis_index("s") * 2 + lax.axis_index("c")

    @pl.when(wid < NCHUNK)
    def _():
        nb = pl.multiple_of(wid * 2048, 8)

        def zero_body(i, _):
            tot_v[pl.ds(pl.multiple_of(i * 16, 16), 16)] = (
                jnp.zeros((16,), jnp.int32))
            return 0
        lax.fori_loop(0, 2048 // 16, zero_body, 0)

        def k_body(k, _):
            pltpu.sync_copy(hist_hbm.at[k, pl.ds(nb, 2048)], acc_v)

            def add_body(i, _):
                sl = pl.ds(pl.multiple_of(i * 16, 16), 16)
                tot_v[sl] = tot_v[sl] + acc_v[sl]
                return 0
            lax.fori_loop(0, 2048 // 16, add_body, 0)
            return 0
        lax.fori_loop(0, NW, k_body, 0)

        def scan_body(i, carry):
            sl = pl.ds(pl.multiple_of(i * 16, 16), 16)
            t16 = tot_v[sl]
            pre = plsc.cumsum(t16)
            tot_v[sl] = carry + (pre - t16)
            return carry + jnp.sum(jnp.where(_lane() == 15, pre, 0))
        total = lax.fori_loop(0, 2048 // 16, scan_body, jnp.int32(0))
        pltpu.sync_copy(tot_v, colpre_hbm.at[pl.ds(nb, 2048)])
        acc_v[pl.ds(0, 16)] = jnp.full((16,), total, jnp.int32)
        pltpu.sync_copy(acc_v.at[pl.ds(0, 16)], csum_hbm.at[wid])


def _s3_place_body(dst_hbm, src_hbm, ea_hbm, hist_hbm, colpre_hbm, csum_hbm,
                   ssrc_hbm, sea_hbm, rs_hbm,
                   next_v, cs_v, co_v, nb_v, hb_v, d_v, s_v, e_v,
                   st_idx, st_src, st_ea, sem):
    wid = lax.axis_index("s") * 2 + lax.axis_index("c")

    pltpu.sync_copy(csum_hbm, cs_v)
    zi = jnp.zeros((16,), jnp.int32)
    ga = plsc.load_gather(cs_v, [_lane(), zi])
    gb = plsc.load_gather(cs_v, [_lane() + 16, zi])
    pa = plsc.cumsum(ga)
    pb = plsc.cumsum(gb)
    suma = jnp.sum(jnp.where(_lane() == 15, pa, 0))
    co_v[pl.ds(0, 16)] = pa - ga
    co_v[pl.ds(16, 16)] = (pb - gb) + suma

    def build_chunk(c, _):
        nb = pl.multiple_of(c * 2048, 8)
        pltpu.sync_copy(colpre_hbm.at[pl.ds(nb, 2048)], nb_v)
        co = _extract_i(co_v, c)

        def addk(k, _):
            pltpu.sync_copy(hist_hbm.at[k, pl.ds(nb, 2048)], hb_v)

            def add_body(i, _):
                sl = pl.ds(pl.multiple_of(i * 16, 16), 16)
                nb_v[sl] = nb_v[sl] + hb_v[sl]
                return 0
            lax.fori_loop(0, 2048 // 16, add_body, 0)
            return 0
        lax.fori_loop(0, wid, addk, 0)

        def store_body(i, _):
            sl = pl.ds(pl.multiple_of(i * 16, 16), 16)
            next_v[pl.ds(pl.multiple_of(nb + i * 16, 16), 16)] = (
                nb_v[sl] + co)
            return 0
        lax.fori_loop(0, 2048 // 16, store_body, 0)
        return 0
    lax.fori_loop(0, NCHUNK, build_chunk, 0)

    @pl.when(wid == 0)
    def _write_rs():
        pltpu.sync_copy(next_v, rs_hbm)

    for c in range(EWP // DCH):
        base = pl.multiple_of(wid * EWP + c * DCH, 8)
        pltpu.sync_copy(dst_hbm.at[pl.ds(base, DCH)], d_v)
        pltpu.sync_copy(src_hbm.at[pl.ds(base, DCH)], s_v)
        pltpu.sync_copy(ea_hbm.at[pl.ds(base, DCH)], e_v)

        def vec_body(i, _):
            sl = pl.ds(pl.multiple_of(i * 16, 16), 16)
            d16 = d_v[sl]
            occ, last = plsc.scan_count(d16)
            b16 = plsc.load_gather(next_v, [d16])
            plsc.store_scatter(next_v, [d16], b16 + occ, mask=last)
            pos = b16 + occ - 1
            r = i // 8
            cl = pl.ds(pl.multiple_of((i % 8) * 16, 16), 16)
            st_idx[r, cl] = pos
            st_src[r, cl] = s_v[sl]
            st_ea[r, cl] = e_v[sl]
            return 0
        lax.fori_loop(0, DCH // 16, vec_body, 0)

        handles = []
        for r in range(DCH // 128):
            handles.append(pltpu.async_copy(
                st_src.at[r], ssrc_hbm.at[st_idx.at[r]], sem))
            handles.append(pltpu.async_copy(
                st_ea.at[r], sea_hbm.at[st_idx.at[r]], sem))
        for h in handles:
            h.wait()


def _sc_sort(dst_p, src_p, ea_p):
    mesh = plsc.VectorSubcoreMesh(core_axis_name="c", subcore_axis_name="s")
    cparams = pltpu.CompilerParams(
        needs_layout_passes=False, use_tc_tiling_on_sc=False)
    hist = pl.kernel(
        _s1_hist_body,
        out_type=jax.ShapeDtypeStruct((NW, NP), jnp.int32),
        mesh=mesh, compiler_params=cparams,
        scratch_types=[
            pltpu.VMEM((NP,), jnp.int32),
            pltpu.VMEM((DCH,), jnp.int32),
        ],
    )(dst_p)
    colpre, csum = pl.kernel(
        _s2_prefix_body,
        out_type=(jax.ShapeDtypeStruct((NP,), jnp.int32),
                  jax.ShapeDtypeStruct((NW, 16), jnp.int32)),
        mesh=mesh, compiler_params=cparams,
        scratch_types=[
            pltpu.VMEM((2048,), jnp.int32),
            pltpu.VMEM((2048,), jnp.int32),
        ],
    )(hist)
    ssrc, sea, rs = pl.kernel(
        _s3_place_body,
        out_type=(jax.ShapeDtypeStruct((EP,), jnp.int32),
                  jax.ShapeDtypeStruct((EP,), jnp.float32),
                  jax.ShapeDtypeStruct((NP,), jnp.int32)),
        mesh=mesh, compiler_params=cparams,
        scratch_types=[
            pltpu.VMEM((NP,), jnp.int32),
            pltpu.VMEM((NW, 16), jnp.int32),
            pltpu.VMEM((32,), jnp.int32),
            pltpu.VMEM((2048,), jnp.int32),
            pltpu.VMEM((2048,), jnp.int32),
            pltpu.VMEM((DCH,), jnp.int32),
            pltpu.VMEM((DCH,), jnp.int32),
            pltpu.VMEM((DCH,), jnp.float32),
            pltpu.VMEM((DCH // 128, 128), jnp.int32),
            pltpu.VMEM((DCH // 128, 128), jnp.int32),
            pltpu.VMEM((DCH // 128, 128), jnp.float32),
            pltpu.SemaphoreType.DMA,
        ],
    )(dst_p, src_p, ea_p, hist, colpre, csum)
    return ssrc, sea, rs


def kernel(x, edge_index, edge_attr, params):
    src = edge_index[0]
    dst = edge_index[1]
    ea = edge_attr[:, 0]

    npad = EP - E
    dst_p = jnp.concatenate(
        [dst, jnp.full((npad,), DSENT, jnp.int32)])
    src_p = jnp.concatenate([src, jnp.zeros((npad,), jnp.int32)])
    ea_p = jnp.concatenate([ea, jnp.zeros((npad,), jnp.float32)])
    srcs_p, eas_p, rs_p = _sc_sort(dst_p, src_p, ea_p)

    targets = (jnp.arange(NW + 1, dtype=jnp.int32) *
               jnp.int32(E // NW)).astype(jnp.int32)
    nstart = jnp.searchsorted(rs_p[:N + 1], targets).astype(jnp.int32)
    nstart = nstart.at[NW].set(N)
    nstart_p = jnp.pad(nstart, (0, 48 - (NW + 1)))

    def layer(h, p, heads, out_ch, act):
        xl, xr = _dense_xlxr(h, p["Wl"], p["bl"], p["Wr"], p["br"], act)
        return _gat_layer_sc(
            xl, xr, srcs_p, eas_p, rs_p, nstart_p,
            p["We"][0], p["att"].reshape(-1), p["bias"],
            heads, out_ch)

    x16 = jnp.pad(x, ((0, 0), (0, 1)))
    p1 = dict(params["c1"])
    p1["Wl"] = jnp.pad(params["c1"]["Wl"], ((0, 1), (0, 0)))
    p1["Wr"] = jnp.pad(params["c1"]["Wr"], ((0, 1), (0, 0)))

    h = layer(x16, p1, 8, 16, None)
    h = layer(h, params["c2"], 8, 16, "elu")
    h = layer(h, params["c3"], 8, 16, "elu")
    h = layer(h, params["c4"], 8, 8, "elu")
    return _final_linear(h, params["lin_W"], params["lin_b"])


# speedup vs baseline: 101.6808x; 1.0914x over previous
import functools

import jax
import jax.numpy as jnp
from jax import lax
from jax.experimental import pallas as pl
from jax.experimental.pallas import tpu as pltpu
from jax.experimental.pallas import tpu_sc as plsc

N = 50000
E = 800000
NW = 32
NB = 128
ECH = 256
RING = 2 * ECH
MM_BLK = 1000

def _lane():
    return lax.iota(jnp.int32, 16)


def _matmul2_kernel(x_ref, wl_ref, bl_ref, wr_ref, br_ref, xl_ref, xr_ref,
                    *, act):
    x = x_ref[...]
    if act == "elu":
        x = jnp.where(x > 0, x, jnp.exp(x) - 1.0)
    xl_ref[...] = jnp.dot(x, wl_ref[...],
                          preferred_element_type=jnp.float32) + bl_ref[...]
    xr_ref[...] = jnp.dot(x, wr_ref[...],
                          preferred_element_type=jnp.float32) + br_ref[...]


def _dense_xlxr(x, wl, bl, wr, br, act):
    n, k = x.shape
    f = wl.shape[1]
    return pl.pallas_call(
        functools.partial(_matmul2_kernel, act=act),
        grid=(n // MM_BLK,),
        in_specs=[
            pl.BlockSpec((MM_BLK, k), lambda i: (i, 0)),
            pl.BlockSpec((k, f), lambda i: (0, 0)),
            pl.BlockSpec((f,), lambda i: (0,)),
            pl.BlockSpec((k, f), lambda i: (0, 0)),
            pl.BlockSpec((f,), lambda i: (0,)),
        ],
        out_specs=[
            pl.BlockSpec((MM_BLK, f), lambda i: (i, 0)),
            pl.BlockSpec((MM_BLK, f), lambda i: (i, 0)),
        ],
        out_shape=[
            jax.ShapeDtypeStruct((n + NB, f), jnp.float32),
            jax.ShapeDtypeStruct((n + NB, f), jnp.float32),
        ],
    )(x, wl, bl, wr, br)


def _final_kernel(h_ref, w_ref, b_ref, o_ref):
    o_ref[...] = jnp.dot(h_ref[...], w_ref[...],
                         preferred_element_type=jnp.float32) + b_ref[...]


def _final_linear(h, w, b):
    n, k = h.shape
    return pl.pallas_call(
        _final_kernel,
        grid=(n // MM_BLK,),
        in_specs=[
            pl.BlockSpec((MM_BLK, k), lambda i: (i, 0)),
            pl.BlockSpec((k, 1), lambda i: (0, 0)),
            pl.BlockSpec((1,), lambda i: (0,)),
        ],
        out_specs=pl.BlockSpec((MM_BLK, 1), lambda i: (i, 0)),
        out_shape=jax.ShapeDtypeStruct((n, 1), jnp.float32),
    )(h, w, b)


def _extract_i(ref, j):
    base = pl.multiple_of((j // 16) * 16, 16)
    v = ref[pl.ds(base, 16)]
    return jnp.sum(jnp.where(_lane() == (j - base), v, 0))


def _extract_f(ref, j):
    base = pl.multiple_of((j // 16) * 16, 16)
    v = ref[pl.ds(base, 16)]
    return jnp.sum(jnp.where(_lane() == (j - base), v, 0.0))


def _make_edge_kernel(nvec, hpv):
    def body(xl_hbm, xr_hbm, src_hbm, ea_hbm, rs_hbm, nstart_hbm,
             we_hbm, att_hbm, bias_hbm, out_hbm,
             ns_v, rs0_v, rs_blk, xr_blk, out_blk, src_ring, ea_ring,
             xj_ring, we_v, att_v, bias_v, gsem):
        wid = lax.axis_index("s") * 2 + lax.axis_index("c")

        pltpu.sync_copy(we_hbm, we_v)
        pltpu.sync_copy(att_hbm, att_v)
        pltpu.sync_copy(bias_hbm, bias_v)
        pltpu.sync_copy(nstart_hbm, ns_v)
        ns = _extract_i(ns_v, wid)
        ne = _extract_i(ns_v, wid + 1)
        nnodes = ne - ns

        @pl.when(nnodes > 0)
        def _worker():
            rsb = pl.multiple_of((ns // 8) * 8, 8)
            pltpu.sync_copy(rs_hbm.at[pl.ds(rsb, 16)], rs0_v)
            eb = _extract_i(rs0_v, ns - rsb)
            cb0 = pl.multiple_of((eb // 8) * 8, 8)

            def stage(i):
                slot = pl.multiple_of((i % 2) * ECH, 8)
                base = pl.multiple_of(cb0 + i * ECH, 8)
                pltpu.sync_copy(src_hbm.at[pl.ds(base, ECH)],
                                src_ring.at[pl.ds(slot, ECH)])
                pltpu.sync_copy(ea_hbm.at[pl.ds(base, ECH)],
                                ea_ring.at[pl.ds(slot, ECH)])
                return pltpu.async_copy(
                    xl_hbm.at[src_ring.at[pl.ds(slot, ECH)]],
                    xj_ring.at[pl.ds(slot, ECH)], gsem)

            def drain_one():
                pltpu.make_async_copy(
                    xl_hbm.at[pl.ds(0, ECH)],
                    xj_ring.at[pl.ds(0, ECH)], gsem).wait()

            stage(0)
            drain_one()
            stage(1)

            zero16 = jnp.zeros((16,), jnp.float32)
            nblk = (nnodes + (NB - 1)) // NB

            def blk_body(b, ci):
                nbase = ns + b * NB
                nb_al = pl.multiple_of((nbase // 8) * 8, 8)
                d0 = nbase - nb_al
                pltpu.sync_copy(xr_hbm.at[pl.ds(nb_al, NB + 8)], xr_blk)
                pltpu.sync_copy(rs_hbm.at[pl.ds(nb_al, NB + 16)], rs_blk)

                def node_body(nloc, ci):
                    s = _extract_i(rs_blk, d0 + nloc)
                    t = _extract_i(rs_blk, d0 + nloc + 1)

                    def edge_body(e, carry):
                        ci = carry[0]
                        accs = carry[1:]
                        need = (e - cb0) // ECH

                        @pl.when(need != ci)
                        def _advance():
                            drain_one()
                            stage(ci + 2)

                        iring = (e - cb0) % RING
                        ea_b = plsc.load_gather(
                            ea_ring, [jnp.full((16,), iring, jnp.int32)])
                        new_num = []
                        new_den = []
                        for v in range(nvec):
                            sl = pl.ds(v * 16, 16)
                            xj = xj_ring[iring, sl]
                            m = (xj + xr_blk[d0 + nloc, sl]
                                 + ea_b * we_v[pl.ds(v * 16, 16)])
                            m = jnp.maximum(m, 0.2 * m)
                            pre = plsc.cumsum(m * att_v[pl.ds(v * 16, 16)])
                            if hpv == 1:
                                exv = jnp.exp(jnp.full((16,), pre[15]))
                            else:
                                lo = pre[7]
                                exv = jnp.exp(jnp.where(
                                    _lane() < 8, lo, pre[15] - lo))
                            new_num.append(accs[v] + exv * xj)
                            new_den.append(accs[nvec + v] + exv)
                        return tuple([need] + new_num + new_den)

                    init = tuple([ci] + [zero16] * (2 * nvec))
                    res = lax.fori_loop(s, t, edge_body, init)
                    for v in range(nvec):
                        sl = pl.ds(v * 16, 16)
                        out_blk[nloc, sl] = (
                            res[1 + v] / (res[1 + nvec + v] + 1e-16)
                            + bias_v[sl])
                    return res[0]

                nmax = jnp.minimum(NB, ne - nbase)
                ci = lax.fori_loop(0, nmax, node_body, ci)

                off = jnp.int32(0)
                for bit in range(7, -1, -1):
                    w = 1 << bit
                    hit = (nmax & w) != 0
                    off_c = off

                    @pl.when(hit)
                    def _flush(off_c=off_c, w=w):
                        pltpu.sync_copy(
                            out_blk.at[pl.ds(off_c, w)],
                            out_hbm.at[pl.ds(nbase + off_c, w)])

                    off = off + jnp.where(hit, w, 0)
                return ci

            lax.fori_loop(0, nblk, blk_body, jnp.int32(0))
            drain_one()

    return body


def _gat_layer_sc(xl, xr, srcs, eas, rs, nstart, wev, attv, biasv,
                  heads, out_ch):
    f = heads * out_ch
    nvec = f // 16
    hpv = 16 // out_ch
    mesh = plsc.VectorSubcoreMesh(core_axis_name="c", subcore_axis_name="s")
    kern = pl.kernel(
        _make_edge_kernel(nvec, hpv),
        out_type=jax.ShapeDtypeStruct((N + NB, nvec * 16), jnp.float32),
        mesh=mesh,
        compiler_params=pltpu.CompilerParams(
            needs_layout_passes=False, use_tc_tiling_on_sc=False),
        scratch_types=[
            pltpu.VMEM((48,), jnp.int32),
            pltpu.VMEM((16,), jnp.int32),
            pltpu.VMEM((NB + 16,), jnp.int32),
            pltpu.VMEM((NB + 8, nvec * 16), jnp.float32),
            pltpu.VMEM((NB, nvec * 16), jnp.float32),
            pltpu.VMEM((RING,), jnp.int32),
            pltpu.VMEM((RING,), jnp.float32),
            pltpu.VMEM((RING, nvec * 16), jnp.float32),
            pltpu.VMEM((nvec * 16,), jnp.float32),
            pltpu.VMEM((nvec * 16,), jnp.float32),
            pltpu.VMEM((nvec * 16,), jnp.float32),
            pltpu.SemaphoreType.DMA,
        ],
    )
    out = kern(xl, xr, srcs, eas, rs, nstart, wev, attv, biasv)
    return out[:N]


NP = 53248
NCHUNK = NP // 2048
EWP = 25088
EP = NW * EWP
DCH = 1792
DSENT = 52000


def _s1_hist_body(dst_hbm, hist_hbm, hist_v, dst_v):
    wid = lax.axis_index("s") * 2 + lax.axis_index("c")

    def zero_body(i, _):
        hist_v[pl.ds(pl.multiple_of(i * 16, 16), 16)] = (
            jnp.zeros((16,), jnp.int32))
        return 0
    lax.fori_loop(0, NP // 16, zero_body, 0)

    def chunk_body(c, _):
        base = pl.multiple_of(wid * EWP + c * DCH, 8)
        pltpu.sync_copy(dst_hbm.at[pl.ds(base, DCH)], dst_v)

        def vec_body(i, _):
            d16 = dst_v[pl.ds(pl.multiple_of(i * 16, 16), 16)]
            occ, last = plsc.scan_count(d16)
            cur = plsc.load_gather(hist_v, [d16])
            plsc.store_scatter(hist_v, [d16], cur + occ, mask=last)
            return 0
        lax.fori_loop(0, DCH // 16, vec_body, 0)
        return 0
    lax.fori_loop(0, EWP // DCH, chunk_body, 0)
    pltpu.sync_copy(hist_v, hist_hbm.at[wid])


def _s2_prefix_body(hist_hbm, colpre_hbm, csum_hbm, base_hbm, tot_v, acc_v):
    wid = lax.axis_index("s") * 2 + lax.axis_index("c")

    @pl.when(wid < NCHUNK)
    def _():
        nb = pl.multiple_of(wid * 2048, 8)

        def zero_body(i, _):
            tot_v[pl.ds(pl.multiple_of(i * 16, 16), 16)] = (
                jnp.zeros((16,), jnp.int32))
            return 0
        lax.fori_loop(0, 2048 // 16, zero_body, 0)

        def k_body(k, _):
            pltpu.sync_copy(tot_v, base_hbm.at[k, pl.ds(nb, 2048)])
            pltpu.sync_copy(hist_hbm.at[k, pl.ds(nb, 2048)], acc_v)

            def add_body(i, _):
                sl = pl.ds(pl.multiple_of(i * 16, 16), 16)
                tot_v[sl] = tot_v[sl] + acc_v[sl]
                return 0
            lax.fori_loop(0, 2048 // 16, add_body, 0)
            return 0
        lax.fori_loop(0, NW, k_body, 0)

        def scan_body(i, carry):
            sl = pl.ds(pl.multiple_of(i * 16, 16), 16)
            t16 = tot_v[sl]
            pre = plsc.cumsum(t16)
            tot_v[sl] = carry + (pre - t16)
            return carry + jnp.sum(jnp.where(_lane() == 15, pre, 0))
        total = lax.fori_loop(0, 2048 // 16, scan_body, jnp.int32(0))
        pltpu.sync_copy(tot_v, colpre_hbm.at[pl.ds(nb, 2048)])
        acc_v[pl.ds(0, 16)] = jnp.full((16,), total, jnp.int32)
        pltpu.sync_copy(acc_v.at[pl.ds(0, 16)], csum_hbm.at[wid])


def _s3_place_body(dst_hbm, src_hbm, ea_hbm, colpre_hbm, csum_hbm, base_hbm,
                   ssrc_hbm, sea_hbm, rs_hbm,
                   next_v, cs_v, co_v, nb_v, hb_v, d_v, s_v, e_v,
                   st_idx, st_src, st_ea, sem):
    wid = lax.axis_index("s") * 2 + lax.axis_index("c")

    pltpu.sync_copy(csum_hbm, cs_v)
    zi = jnp.zeros((16,), jnp.int32)
    ga = plsc.load_gather(cs_v, [_lane(), zi])
    gb = plsc.load_gather(cs_v, [_lane() + 16, zi])
    pa = plsc.cumsum(ga)
    pb = plsc.cumsum(gb)
    suma = jnp.sum(jnp.where(_lane() == 15, pa, 0))
    co_v[pl.ds(0, 16)] = pa - ga
    co_v[pl.ds(16, 16)] = (pb - gb) + suma

    def build_chunk(c, _):
        nb = pl.multiple_of(c * 2048, 8)
        pltpu.sync_copy(colpre_hbm.at[pl.ds(nb, 2048)], nb_v)
        pltpu.sync_copy(base_hbm.at[wid, pl.ds(nb, 2048)], hb_v)
        co = _extract_i(co_v, c)

        def store_body(i, _):
            sl = pl.ds(pl.multiple_of(i * 16, 16), 16)
            next_v[pl.ds(pl.multiple_of(nb + i * 16, 16), 16)] = (
                nb_v[sl] + hb_v[sl] + co)
            return 0
        lax.fori_loop(0, 2048 // 16, store_body, 0)
        return 0
    lax.fori_loop(0, NCHUNK, build_chunk, 0)

    @pl.when(wid == 0)
    def _write_rs():
        pltpu.sync_copy(next_v, rs_hbm)

    pending = []
    for c in range(EWP // DCH):
        p = c % 2
        base = pl.multiple_of(wid * EWP + c * DCH, 8)
        pltpu.sync_copy(dst_hbm.at[pl.ds(base, DCH)], d_v)
        pltpu.sync_copy(src_hbm.at[pl.ds(base, DCH)], s_v)
        pltpu.sync_copy(ea_hbm.at[pl.ds(base, DCH)], e_v)

        def vec_body(i, _, p=p):
            sl = pl.ds(pl.multiple_of(i * 16, 16), 16)
            d16 = d_v[sl]
            occ, last = plsc.scan_count(d16)
            b16 = plsc.load_gather(next_v, [d16])
            plsc.store_scatter(next_v, [d16], b16 + occ, mask=last)
            pos = b16 + occ - 1
            r = i // 8
            cl = pl.ds(pl.multiple_of((i % 8) * 16, 16), 16)
            st_idx[p, r, cl] = pos
            st_src[p, r, cl] = s_v[sl]
            st_ea[p, r, cl] = e_v[sl]
            return 0
        lax.fori_loop(0, DCH // 16, vec_body, 0)

        for h in pending:
            h.wait()
        pending = []
        for r in range(DCH // 128):
            pending.append(pltpu.async_copy(
                st_src.at[p, r], ssrc_hbm.at[st_idx.at[p, r]], sem))
            pending.append(pltpu.async_copy(
                st_ea.at[p, r], sea_hbm.at[st_idx.at[p, r]], sem))
    for h in pending:
        h.wait()


def _sc_sort(dst_p, src_p, ea_p):
    mesh = plsc.VectorSubcoreMesh(core_axis_name="c", subcore_axis_name="s")
    cparams = pltpu.CompilerParams(
        needs_layout_passes=False, use_tc_tiling_on_sc=False)
    hist = pl.kernel(
        _s1_hist_body,
        out_type=jax.ShapeDtypeStruct((NW, NP), jnp.int32),
        mesh=mesh, compiler_params=cparams,
        scratch_types=[
            pltpu.VMEM((NP,), jnp.int32),
            pltpu.VMEM((DCH,), jnp.int32),
        ],
    )(dst_p)
    colpre, csum, basew = pl.kernel(
        _s2_prefix_body,
        out_type=(jax.ShapeDtypeStruct((NP,), jnp.int32),
                  jax.ShapeDtypeStruct((NW, 16), jnp.int32),
                  jax.ShapeDtypeStruct((NW, NP), jnp.int32)),
        mesh=mesh, compiler_params=cparams,
        scratch_types=[
            pltpu.VMEM((2048,), jnp.int32),
            pltpu.VMEM((2048,), jnp.int32),
        ],
    )(hist)
    ssrc, sea, rs = pl.kernel(
        _s3_place_body,
        out_type=(jax.ShapeDtypeStruct((EP,), jnp.int32),
                  jax.ShapeDtypeStruct((EP,), jnp.float32),
                  jax.ShapeDtypeStruct((NP,), jnp.int32)),
        mesh=mesh, compiler_params=cparams,
        scratch_types=[
            pltpu.VMEM((NP,), jnp.int32),
            pltpu.VMEM((NW, 16), jnp.int32),
            pltpu.VMEM((32,), jnp.int32),
            pltpu.VMEM((2048,), jnp.int32),
            pltpu.VMEM((2048,), jnp.int32),
            pltpu.VMEM((DCH,), jnp.int32),
            pltpu.VMEM((DCH,), jnp.int32),
            pltpu.VMEM((DCH,), jnp.float32),
            pltpu.VMEM((2, DCH // 128, 128), jnp.int32),
            pltpu.VMEM((2, DCH // 128, 128), jnp.int32),
            pltpu.VMEM((2, DCH // 128, 128), jnp.float32),
            pltpu.SemaphoreType.DMA,
        ],
    )(dst_p, src_p, ea_p, colpre, csum, basew)
    return ssrc, sea, rs


def kernel(x, edge_index, edge_attr, params):
    src = edge_index[0]
    dst = edge_index[1]
    ea = edge_attr[:, 0]

    npad = EP - E
    dst_p = jnp.concatenate(
        [dst, jnp.full((npad,), DSENT, jnp.int32)])
    src_p = jnp.concatenate([src, jnp.zeros((npad,), jnp.int32)])
    ea_p = jnp.concatenate([ea, jnp.zeros((npad,), jnp.float32)])
    srcs_p, eas_p, rs_p = _sc_sort(dst_p, src_p, ea_p)

    targets = (jnp.arange(NW + 1, dtype=jnp.int32) *
               jnp.int32(E // NW)).astype(jnp.int32)
    nstart = jnp.searchsorted(rs_p[:N + 1], targets).astype(jnp.int32)
    nstart = nstart.at[NW].set(N)
    nstart_p = jnp.pad(nstart, (0, 48 - (NW + 1)))

    def layer(h, p, heads, out_ch, act):
        xl, xr = _dense_xlxr(h, p["Wl"], p["bl"], p["Wr"], p["br"], act)
        return _gat_layer_sc(
            xl, xr, srcs_p, eas_p, rs_p, nstart_p,
            p["We"][0], p["att"].reshape(-1), p["bias"],
            heads, out_ch)

    x16 = jnp.pad(x, ((0, 0), (0, 1)))
    p1 = dict(params["c1"])
    p1["Wl"] = jnp.pad(params["c1"]["Wl"], ((0, 1), (0, 0)))
    p1["Wr"] = jnp.pad(params["c1"]["Wr"], ((0, 1), (0, 0)))

    h = layer(x16, p1, 8, 16, None)
    h = layer(h, params["c2"], 8, 16, "elu")
    h = layer(h, params["c3"], 8, 16, "elu")
    h = layer(h, params["c4"], 8, 8, "elu")
    return _final_linear(h, params["lin_W"], params["lin_b"])
